# baseline (speedup 1.0000x reference)
import functools
import jax
import jax.numpy as jnp
from jax import lax
from jax.experimental import pallas as pl
from jax.experimental.pallas import tpu as pltpu

K = 7
PS = 3
WS = 15
EPS = 1e-8
R = WS // 2
O = WS * WS
H = W = 128
CE = 8
C = 3
NEG_BIG = -1.0e30


def _rollw(a, s):
    if s % W == 0:
        return a
    return jnp.roll(a, s, axis=-1)


def _boxw(a):
    return a + _rollw(a, 1) + _rollw(a, -1)


def _nnn_kernel(xe2_ref, ye2_ref, x2_ref, lt_ref, out_ref, u_ref, bx_ref,
                cc_ref):
    f32 = jnp.float32

    lt = lt_ref[0]
    blt = _boxw(lt[0:H] + lt[1:H + 1] + lt[2:H + 2])
    invt = -1.0 / (jnp.exp(blt * (1.0 / (PS * PS))) + EPS)

    ny = jnp.zeros((H + 2, W), f32)
    for c in range(CE):
        yc = ye2_ref[c, 0:H + 2, :]
        ny = ny + yc * yc
    by = _boxw(ny[0:H] + ny[1:H + 1] + ny[2:H + 2])

    nx = jnp.zeros((H + 16, W), f32)
    for c in range(CE):
        xc = xe2_ref[c, 0:H + 16, :]
        nx = nx + xc * xc
    bx_ref[...] = _boxw(nx[0:H + 14] + nx[1:H + 15] + nx[2:H + 16])

    def dy_body(i, m):
        def row_body(rt, _):
            b = rt * 8
            yv = [ye2_ref[c, pl.ds(b, 8), :] for c in range(CE)]
            xv = [xe2_ref[c, pl.ds(i + b, 8), :] for c in range(CE)]
            for dxk in range(WS):
                dx = dxk - R
                acc = yv[0] * _rollw(xv[0], -dx)
                for c in range(1, CE):
                    acc = acc + yv[c] * _rollw(xv[c], -dx)
                cc_ref[dxk, pl.ds(b, 8), :] = acc
            return 0

        lax.fori_loop(0, 17, row_body, 0)

        bxs_rows = bx_ref[pl.ds(i, H), :]
        for dxk in range(WS):
            dx = dxk - R
            vc = (cc_ref[dxk, 0:H, :] + cc_ref[dxk, 1:H + 1, :]
                  + cc_ref[dxk, 2:H + 2, :])
            bc = _boxw(vc)
            d = by + _rollw(bxs_rows, -dx) - 2.0 * bc
            lg = d * invt
            if dxk == R:
                lg = jnp.where(i == R, NEG_BIG, lg)
            u_ref[pl.ds(i * WS + dxk, 1)] = lg[None]
            m = jnp.maximum(m, lg)
        return m

    m = lax.fori_loop(0, WS, dy_body, jnp.full((H, W), -3.0e38, f32))

    def exp_body(o, s):
        u = jnp.exp(u_ref[pl.ds(o, 1)] - m[None])
        u_ref[pl.ds(o, 1)] = u
        return s + u[0]

    s = lax.fori_loop(0, O, exp_body, jnp.zeros((H, W), f32))

    for j in range(K):
        last = j == K - 1

        def agg_body(i, accs, last=last):
            xh = [x2_ref[c, pl.ds(i, H), :] for c in range(C)]
            a0, a1, a2, sn = accs
            for dxk in range(WS):
                dx = dxk - R
                o = i * WS + dxk
                u = u_ref[pl.ds(o, 1)][0]
                w = u / s
                a0 = a0 + w * _rollw(xh[0], -dx)
                a1 = a1 + w * _rollw(xh[1], -dx)
                a2 = a2 + w * _rollw(xh[2], -dx)
                if not last:
                    un = u * (1.0 + (EPS - w))
                    u_ref[pl.ds(o, 1)] = un[None]
                    sn = sn + un
            return (a0, a1, a2, sn)

        z = jnp.zeros((H, W), f32)
        acc = lax.fori_loop(0, WS, agg_body, (z, z, z, z))
        for c in range(C):
            out_ref[j * C + c] = acc[c] - x2_ref[c, pl.ds(R, H), :]
        s = acc[3]


@jax.jit
def _run(x, xe, ye, log_temp):
    x0 = x[0]
    xe0 = xe[0]
    ye0 = ye[0]
    lt0 = log_temp[0]

    xe2 = jnp.concatenate([xe0[:, -8:, :], xe0, xe0[:, :16, :]], axis=1)
    ye2 = jnp.concatenate([ye0[:, -1:, :], ye0, ye0[:, :7, :]], axis=1)
    x2 = jnp.concatenate([x0[:, -R:, :], x0, x0[:, :R, :]], axis=1)
    lt2 = jnp.concatenate([lt0[:, -1:, :], lt0, lt0[:, :1, :]], axis=1)

    z = pl.pallas_call(
        _nnn_kernel,
        out_shape=jax.ShapeDtypeStruct((K * C, H, W), jnp.float32),
        scratch_shapes=[
            pltpu.VMEM((O, H, W), jnp.float32),
            pltpu.VMEM((H + 14, W), jnp.float32),
            pltpu.VMEM((WS, H + 8, W), jnp.float32),
        ],
    )(xe2, ye2, x2, lt2)

    out = jnp.concatenate([x0, z], axis=0)[None]
    return out


def kernel(x, xe, ye, log_temp):
    return _run(x, xe, ye, log_temp)


# speedup vs baseline: 1.0341x; 1.0341x over previous
import functools
import jax
import jax.numpy as jnp
from jax import lax
from jax.experimental import pallas as pl
from jax.experimental.pallas import tpu as pltpu

K = 7
PS = 3
WS = 15
EPS = 1e-8
R = WS // 2
O = WS * WS
H = W = 128
CE = 8
C = 3
NEG_BIG = -1.0e30


def _rollw(a, s):
    if s % W == 0:
        return a
    return jnp.roll(a, s, axis=-1)


def _boxw(a):
    return a + _rollw(a, 1) + _rollw(a, -1)


def _nnn_kernel(xe2_ref, ye2_ref, x2_ref, lt_ref, out_ref, u_ref, bx_ref):
    f32 = jnp.float32

    lt = lt_ref[0]
    blt = _boxw(lt[0:H] + lt[1:H + 1] + lt[2:H + 2])
    invt = -1.0 / (jnp.exp(blt * (1.0 / (PS * PS))) + EPS)

    ny = jnp.zeros((H + 2, W), f32)
    for c in range(CE):
        yc = ye2_ref[c]
        ny = ny + yc * yc
    by = _boxw(ny[0:H] + ny[1:H + 1] + ny[2:H + 2])

    nx = jnp.zeros((H + 16, W), f32)
    for c in range(CE):
        xc = xe2_ref[c]
        nx = nx + xc * xc
    bx_ref[...] = _boxw(nx[0:H + 14] + nx[1:H + 15] + nx[2:H + 16])

    def dy_body(i, m):
        xh = [xe2_ref[c, pl.ds(i, H + 2), :] for c in range(CE)]
        yv = [ye2_ref[c] for c in range(CE)]
        bxs_rows = bx_ref[pl.ds(i, H), :]
        for dxk in range(WS):
            dx = dxk - R
            cc = jnp.zeros((H + 2, W), f32)
            for c in range(CE):
                cc = cc + yv[c] * _rollw(xh[c], -dx)
            bc = _boxw(cc[0:H] + cc[1:H + 1] + cc[2:H + 2])
            d = by + _rollw(bxs_rows, -dx) - 2.0 * bc
            lg = d * invt
            if dxk == R:
                lg = jnp.where(i == R, NEG_BIG, lg)
            u_ref[pl.ds(i * WS + dxk, 1)] = lg[None]
            m = jnp.maximum(m, lg)
        return m

    m = lax.fori_loop(0, WS, dy_body, jnp.full((H, W), -3.0e38, f32))

    def exp_body(o, s):
        u = jnp.exp(u_ref[pl.ds(o, 1)] - m[None])
        u_ref[pl.ds(o, 1)] = u
        return s + u[0]

    s = lax.fori_loop(0, O, exp_body, jnp.zeros((H, W), f32))

    for j in range(K):
        last = j == K - 1

        rs = 1.0 / s

        def agg_body(i, accs, last=last):
            xh = [x2_ref[c, pl.ds(i, H), :] for c in range(C)]
            a0, a1, a2, sn = accs
            for dxk in range(WS):
                dx = dxk - R
                o = i * WS + dxk
                u = u_ref[pl.ds(o, 1)][0]
                w = u * rs
                a0 = a0 + w * _rollw(xh[0], -dx)
                a1 = a1 + w * _rollw(xh[1], -dx)
                a2 = a2 + w * _rollw(xh[2], -dx)
                if not last:
                    un = u * (1.0 + (EPS - w))
                    u_ref[pl.ds(o, 1)] = un[None]
                    sn = sn + un
            return (a0, a1, a2, sn)

        z = jnp.zeros((H, W), f32)
        acc = lax.fori_loop(0, WS, agg_body, (z, z, z, z))
        for c in range(C):
            out_ref[j * C + c] = acc[c] - x2_ref[c, pl.ds(R, H), :]
        s = acc[3]


@jax.jit
def _run(x, xe, ye, log_temp):
    x0 = x[0]
    xe0 = xe[0]
    ye0 = ye[0]
    lt0 = log_temp[0]

    xe2 = jnp.concatenate([xe0[:, -8:, :], xe0, xe0[:, :8, :]], axis=1)
    ye2 = jnp.concatenate([ye0[:, -1:, :], ye0, ye0[:, :1, :]], axis=1)
    x2 = jnp.concatenate([x0[:, -R:, :], x0, x0[:, :R, :]], axis=1)
    lt2 = jnp.concatenate([lt0[:, -1:, :], lt0, lt0[:, :1, :]], axis=1)

    z = pl.pallas_call(
        _nnn_kernel,
        out_shape=jax.ShapeDtypeStruct((K * C, H, W), jnp.float32),
        scratch_shapes=[
            pltpu.VMEM((O, H, W), jnp.float32),
            pltpu.VMEM((H + 14, W), jnp.float32),
        ],
    )(xe2, ye2, x2, lt2)

    out = jnp.concatenate([x0, z], axis=0)[None]
    return out


def kernel(x, xe, ye, log_temp):
    return _run(x, xe, ye, log_temp)
